# Initial kernel scaffold; baseline (speedup 1.0000x reference)
import jax
import jax.numpy as jnp
from jax.experimental import pallas as pl


def kernel(x, edge_index, edge_attr, Wl0, bl0, Wr0, br0, We0, att0, bo0, Wl1, bl1, Wr1, br1, We1, att1, bo1, Wl2, bl2, Wr2, br2, We2, att2, bo2, D1W, D1b, D2W, D2b):
    raise NotImplementedError("write your pallas kernel here")


# speedup vs baseline: 1.9164x; 1.9164x over previous
import functools

import jax
import jax.numpy as jnp
from jax import lax
from jax.experimental import pallas as pl
from jax.experimental.pallas import tpu as pltpu
from jax.experimental.pallas import tpu_sc as plsc

N = 10000
E = 160000
H = 4
C = 128
HC = H * C

NC = 2
NS = 16
NW = NC * NS
EPW = E // NW
EB = 40
NITER = EPW // EB

NPAD = 40960
COLS_PER_TILE = NPAD // NS
RED_CHUNK = 512

_MESH = plsc.VectorSubcoreMesh(core_axis_name="c", subcore_axis_name="s")

_F32 = jnp.float32
_NEG_INF = float("-inf")


def _wid():
  return lax.axis_index("s") * NC + lax.axis_index("c")


def _mo(v):
  return pl.multiple_of(v, 8)


def _iota16():
  return lax.iota(jnp.int32, 16)


def _lane_shift_gather(buf_ref, vec, idx):
  buf_ref[...] = vec
  return plsc.load_gather(buf_ref, [idx])


def _lane_sum_all(buf_ref, v):
  it = _iota16()
  for d in (1, 2, 4, 8):
    buf_ref[...] = v
    v = v + plsc.load_gather(buf_ref, [it ^ d])
  return v


def _seg_scan_rmw(acc_ref, kbuf_ref, vbuf_ref, key, val, is_add):
  ks, vs = plsc.sort_key_val(key, val)
  it = _iota16()
  kbuf_ref[...] = ks
  for d in (1, 2, 4, 8):
    idx = jnp.maximum(it - d, 0)
    kp = plsc.load_gather(kbuf_ref, [idx])
    vp = _lane_shift_gather(vbuf_ref, vs, idx)
    same = (it >= d) & (kp == ks)
    if is_add:
      vs = vs + jnp.where(same, vp, 0.0)
    else:
      vs = jnp.where(same, jnp.maximum(vs, vp), vs)
  kn = plsc.load_gather(kbuf_ref, [jnp.minimum(it + 1, 15)])
  is_end = (it == 15) | (ks != kn)
  cur = plsc.load_gather(acc_ref, [ks])
  if is_add:
    new = cur + vs
  else:
    new = jnp.maximum(cur, vs)
  plsc.store_scatter(acc_ref, [ks], new, mask=is_end)


def _edge_keys(dst_ref, g):
  it = _iota16()
  rep = plsc.load_gather(dst_ref, [g * 4 + lax.shift_right_logical(it, 2)])
  return rep * 4 + (it & 3)


def _publish_and_reduce(acc_ref, sh_ref, red_ref, ob_ref, out_hbm, is_add):
  sid = lax.axis_index("s")
  cid = lax.axis_index("c")
  pltpu.sync_copy(acc_ref, sh_ref.at[sid])
  plsc.subcore_barrier()
  base = sid * COLS_PER_TILE

  def col_chunk(cc, _):
    pltpu.sync_copy(
        sh_ref.at[:, pl.ds(_mo(base + cc * RED_CHUNK), RED_CHUNK)], red_ref)

    def vec_body(i, _):
      v = red_ref[0, pl.ds(i * 16, 16)]
      for r in range(1, NS):
        w = red_ref[r, pl.ds(i * 16, 16)]
        v = (v + w) if is_add else jnp.maximum(v, w)
      ob_ref[pl.ds(cc * RED_CHUNK + i * 16, 16)] = v
      return 0

    lax.fori_loop(0, RED_CHUNK // 16, vec_body, 0)
    return 0

  lax.fori_loop(0, COLS_PER_TILE // RED_CHUNK, col_chunk, 0)
  pltpu.sync_copy(ob_ref, out_hbm.at[cid, pl.ds(_mo(base), COLS_PER_TILE)])


@functools.partial(
    pl.kernel,
    out_type=jax.ShapeDtypeStruct((E * H,), _F32),
    mesh=_MESH,
    scratch_types=[
        pltpu.VMEM((EB,), jnp.int32),
        pltpu.VMEM((EB,), jnp.int32),
        pltpu.VMEM((EB, HC), _F32),
        pltpu.VMEM((EB, HC), _F32),
        pltpu.VMEM((EB, HC), _F32),
        pltpu.VMEM((HC,), _F32),
        pltpu.VMEM((EB * H,), _F32),
        pltpu.VMEM((16,), _F32),
        pltpu.SemaphoreType.DMA,
    ],
    compiler_params=pltpu.CompilerParams(needs_layout_passes=False),
)
def _sc_logits_kernel(xl_hbm, xr_hbm, ef_hbm, src_hbm, dst_hbm, att_hbm,
                      logits_hbm, srcb, dstb, xlr, xrr, efr,
                      att_v, lbuf, shuf, sem):
  wid = _wid()
  base = wid * EPW
  pltpu.sync_copy(att_hbm, att_v)

  def block(j, _):
    pltpu.sync_copy(src_hbm.at[pl.ds(_mo(base + j * EB), EB)], srcb)
    pltpu.sync_copy(dst_hbm.at[pl.ds(_mo(base + j * EB), EB)], dstb)
    c1 = pltpu.async_copy(xl_hbm.at[srcb], xlr, sem)
    c2 = pltpu.async_copy(xr_hbm.at[dstb], xrr, sem)
    c3 = pltpu.async_copy(ef_hbm.at[pl.ds(_mo(base + j * EB), EB)], efr, sem)
    c1.wait()
    c2.wait()
    c3.wait()

    def edge(e, _):
      it = _iota16()
      tot = jnp.zeros((16,), _F32)
      for hh in range(H):
        acc = jnp.zeros((16,), _F32)
        for v in range(C // 16):
          off = hh * C + v * 16
          z = (xlr[e, pl.ds(off, 16)] + xrr[e, pl.ds(off, 16)]
               + efr[e, pl.ds(off, 16)])
          z = jnp.where(z >= 0.0, z, 0.2 * z)
          acc = acc + z * att_v[pl.ds(off, 16)]
        s = _lane_sum_all(shuf, acc)
        tot = jnp.where(it == hh, s, tot)
      plsc.store_scatter(lbuf, [e * H + (it & 3)], tot, mask=it < H)
      return 0

    lax.fori_loop(0, EB, edge, 0)
    pltpu.sync_copy(lbuf, logits_hbm.at[pl.ds(_mo((base + j * EB) * H), EB * H)])
    return 0

  lax.fori_loop(0, NITER, block, 0)


_LCHUNK = 4000


@functools.partial(
    pl.kernel,
    out_type=jax.ShapeDtypeStruct((NC, NPAD), _F32),
    mesh=_MESH,
    scratch_types=[
        pltpu.VMEM((NPAD,), _F32),
        pltpu.VMEM((EPW + 32,), jnp.int32),
        pltpu.VMEM((_LCHUNK,), _F32),
        pltpu.VMEM((16,), jnp.int32),
        pltpu.VMEM((16,), _F32),
        pltpu.VMEM_SHARED((NS, NPAD), _F32),
        pltpu.VMEM((NS, RED_CHUNK), _F32),
        pltpu.VMEM((COLS_PER_TILE,), _F32),
    ],
    compiler_params=pltpu.CompilerParams(needs_layout_passes=False),
)
def _sc_segmax_kernel(logits_hbm, dst_hbm, mpart_hbm, acc, dst_v, lch,
                      kbuf, vbuf, sh, red, ob):
  wid = _wid()
  pltpu.sync_copy(dst_hbm.at[pl.ds(_mo(wid * EPW), EPW)],
                  dst_v.at[pl.ds(0, EPW)])

  def init(i, _):
    acc[pl.ds(i * 16, 16)] = jnp.full((16,), _NEG_INF, _F32)
    return 0

  lax.fori_loop(0, NPAD // 16, init, 0)

  def stage(cc, _):
    pltpu.sync_copy(
        logits_hbm.at[pl.ds(_mo(wid * EPW * H + cc * _LCHUNK), _LCHUNK)], lch)

    def group(gg, _):
      g = cc * (_LCHUNK // 16) + gg
      key = _edge_keys(dst_v, g)
      val = lch[pl.ds(gg * 16, 16)]
      _seg_scan_rmw(acc, kbuf, vbuf, key, val, is_add=False)
      return 0

    lax.fori_loop(0, _LCHUNK // 16, group, 0)
    return 0

  lax.fori_loop(0, (EPW * H) // _LCHUNK, stage, 0)
  _publish_and_reduce(acc, sh, red, ob, mpart_hbm, is_add=False)


@functools.partial(
    pl.kernel,
    out_type=(jax.ShapeDtypeStruct((E * H,), _F32),
              jax.ShapeDtypeStruct((NC, NPAD), _F32)),
    mesh=_MESH,
    scratch_types=[
        pltpu.VMEM((NPAD,), _F32),
        pltpu.VMEM((COLS_PER_TILE,), _F32),
        pltpu.VMEM((EPW + 32,), jnp.int32),
        pltpu.VMEM((_LCHUNK,), _F32),
        pltpu.VMEM((_LCHUNK,), _F32),
        pltpu.VMEM((80,), jnp.int32),
        pltpu.VMEM((80,), _F32),
        pltpu.VMEM_SHARED((NPAD,), _F32),
    ],
    compiler_params=pltpu.CompilerParams(needs_layout_passes=False),
)
def _sc_exsum_kernel(logits_hbm, dst_hbm, mpart_hbm, ex_hbm, spart_hbm,
                     m_tab, tbuf, dst_v, lch, xch, kb, xb, s_sh):
  wid = _wid()
  sid = lax.axis_index("s")
  cid = lax.axis_index("c")
  pltpu.sync_copy(mpart_hbm.at[0], m_tab)

  def mchunk(k, _):
    pltpu.sync_copy(mpart_hbm.at[1, pl.ds(_mo(k * COLS_PER_TILE), COLS_PER_TILE)],
                    tbuf)

    def mvec(i, _):
      o = k * COLS_PER_TILE + i * 16
      m = jnp.maximum(m_tab[pl.ds(o, 16)], tbuf[pl.ds(i * 16, 16)])
      m_tab[pl.ds(o, 16)] = jnp.where(m < -1e30, 0.0, m)
      return 0

    lax.fori_loop(0, COLS_PER_TILE // 16, mvec, 0)
    return 0

  lax.fori_loop(0, NPAD // COLS_PER_TILE, mchunk, 0)

  def zvec(i, _):
    xch[pl.ds(i * 16, 16)] = jnp.zeros((16,), _F32)
    return 0

  lax.fori_loop(0, COLS_PER_TILE // 16, zvec, 0)
  pltpu.sync_copy(xch.at[pl.ds(0, COLS_PER_TILE)],
                  s_sh.at[pl.ds(_mo(sid * COLS_PER_TILE), COLS_PER_TILE)])
  plsc.subcore_barrier()

  pltpu.sync_copy(dst_hbm.at[pl.ds(_mo(wid * EPW), EPW)], dst_v.at[pl.ds(0, EPW)])

  def stage(cc, _):
    off = _mo(wid * EPW * H + cc * _LCHUNK)
    pltpu.sync_copy(logits_hbm.at[pl.ds(off, _LCHUNK)], lch)

    def bundle(bb, _):
      for q in range(5):
        gg = bb * 5 + q
        g = cc * (_LCHUNK // 16) + gg
        key = _edge_keys(dst_v, g)
        lv = lch[pl.ds(gg * 16, 16)]
        mg = plsc.load_gather(m_tab, [key])
        ex = jnp.exp(lv - mg)
        xch[pl.ds(gg * 16, 16)] = ex
        kb[pl.ds(q * 16, 16)] = key
        xb[pl.ds(q * 16, 16)] = ex
      pltpu.sync_copy(xb, s_sh.at[kb], add=True)
      return 0

    lax.fori_loop(0, _LCHUNK // 80, bundle, 0)
    pltpu.sync_copy(xch, ex_hbm.at[pl.ds(off, _LCHUNK)])
    return 0

  lax.fori_loop(0, (EPW * H) // _LCHUNK, stage, 0)
  plsc.subcore_barrier()
  pltpu.sync_copy(s_sh.at[pl.ds(_mo(sid * COLS_PER_TILE), COLS_PER_TILE)],
                  tbuf)
  pltpu.sync_copy(tbuf, spart_hbm.at[cid, pl.ds(_mo(sid * COLS_PER_TILE),
                                                COLS_PER_TILE)])


@functools.partial(
    pl.kernel,
    out_type=jax.ShapeDtypeStruct((E * H,), _F32),
    mesh=_MESH,
    scratch_types=[
        pltpu.VMEM((NPAD,), _F32),
        pltpu.VMEM((COLS_PER_TILE,), _F32),
        pltpu.VMEM((EPW + 32,), jnp.int32),
        pltpu.VMEM((_LCHUNK,), _F32),
        pltpu.VMEM((_LCHUNK,), _F32),
    ],
    compiler_params=pltpu.CompilerParams(needs_layout_passes=False),
)
def _sc_alpha_kernel(ex_hbm, dst_hbm, spart_hbm, alpha_hbm,
                     s_tab, tbuf, dst_v, xch, ach):
  wid = _wid()
  pltpu.sync_copy(spart_hbm.at[0], s_tab)

  def schunk(k, _):
    pltpu.sync_copy(spart_hbm.at[1, pl.ds(_mo(k * COLS_PER_TILE), COLS_PER_TILE)],
                    tbuf)

    def svec(i, _):
      o = k * COLS_PER_TILE + i * 16
      s_tab[pl.ds(o, 16)] = (s_tab[pl.ds(o, 16)] + tbuf[pl.ds(i * 16, 16)]
                             + 1e-16)
      return 0

    lax.fori_loop(0, COLS_PER_TILE // 16, svec, 0)
    return 0

  lax.fori_loop(0, NPAD // COLS_PER_TILE, schunk, 0)
  pltpu.sync_copy(dst_hbm.at[pl.ds(_mo(wid * EPW), EPW)], dst_v.at[pl.ds(0, EPW)])

  def stage(cc, _):
    off = _mo(wid * EPW * H + cc * _LCHUNK)
    pltpu.sync_copy(ex_hbm.at[pl.ds(off, _LCHUNK)], xch)

    def group(gg, _):
      g = cc * (_LCHUNK // 16) + gg
      key = _edge_keys(dst_v, g)
      sg = plsc.load_gather(s_tab, [key])
      ach[pl.ds(gg * 16, 16)] = xch[pl.ds(gg * 16, 16)] / sg * 0.25
      return 0

    lax.fori_loop(0, _LCHUNK // 16, group, 0)
    pltpu.sync_copy(ach, alpha_hbm.at[pl.ds(off, _LCHUNK)])
    return 0

  lax.fori_loop(0, (EPW * H) // _LCHUNK, stage, 0)


_ZROWS = 32
_NACC = 10240
_NSTRIPE = _NACC // NS
_NLAST = N - (NS - 1) * _NSTRIPE


@functools.partial(
    pl.kernel,
    out_type=jax.ShapeDtypeStruct((NC, N, C), _F32),
    mesh=_MESH,
    scratch_types=[
        pltpu.VMEM((EB,), jnp.int32),
        pltpu.VMEM((EB,), jnp.int32),
        pltpu.VMEM((EB, HC), _F32),
        pltpu.VMEM((EB, C), _F32),
        pltpu.VMEM((EB * H,), _F32),
        pltpu.VMEM((_ZROWS, C), _F32),
        pltpu.VMEM_SHARED((_NACC, C), _F32),
        pltpu.SemaphoreType.DMA,
    ],
    compiler_params=pltpu.CompilerParams(needs_layout_passes=False),
)
def _sc_message_kernel(xl_hbm, src_hbm, dst_hbm, alpha_hbm, out_hbm,
                       srcb, dstb, xlr, msg, abuf, zbuf, acc_sh, sem):
  wid = _wid()
  sid = lax.axis_index("s")
  cid = lax.axis_index("c")
  base = wid * EPW

  for r in range(_ZROWS):
    for v in range(C // 16):
      zbuf[r, pl.ds(v * 16, 16)] = jnp.zeros((16,), _F32)

  def zrow(k, _):
    pltpu.sync_copy(
        zbuf, acc_sh.at[pl.ds(_mo(sid * _NSTRIPE + k * _ZROWS), _ZROWS), :])
    return 0

  lax.fori_loop(0, _NSTRIPE // _ZROWS, zrow, 0)
  plsc.subcore_barrier()

  def block(j, _):
    pltpu.sync_copy(src_hbm.at[pl.ds(_mo(base + j * EB), EB)], srcb)
    pltpu.sync_copy(dst_hbm.at[pl.ds(_mo(base + j * EB), EB)], dstb)
    pltpu.sync_copy(alpha_hbm.at[pl.ds(_mo((base + j * EB) * H), EB * H)], abuf)
    pltpu.async_copy(xl_hbm.at[srcb], xlr, sem).wait()

    def edge(e, _):
      av = []
      for hh in range(H):
        av.append(plsc.load_gather(
            abuf, [jnp.full((16,), e * H + hh, jnp.int32)]))
      for v in range(C // 16):
        m = av[0] * xlr[e, pl.ds(v * 16, 16)]
        for hh in range(1, H):
          m = m + av[hh] * xlr[e, pl.ds(hh * C + v * 16, 16)]
        msg[e, pl.ds(v * 16, 16)] = m
      return 0

    lax.fori_loop(0, EB, edge, 0)
    pltpu.sync_copy(msg, acc_sh.at[dstb], add=True)
    return 0

  lax.fori_loop(0, NITER, block, 0)
  plsc.subcore_barrier()

  @pl.when(sid < NS - 1)
  def _():
    pltpu.sync_copy(acc_sh.at[pl.ds(_mo(sid * _NSTRIPE), _NSTRIPE), :],
                    out_hbm.at[cid, pl.ds(_mo(sid * _NSTRIPE), _NSTRIPE), :])

  @pl.when(sid == NS - 1)
  def _():
    pltpu.sync_copy(acc_sh.at[pl.ds(_mo(sid * _NSTRIPE), _NLAST), :],
                    out_hbm.at[cid, pl.ds(_mo(sid * _NSTRIPE), _NLAST), :])


_PREC = jax.lax.Precision.HIGHEST


def _proj_tc(h, Wl, bl, Wr, br):
  n, din = h.shape
  bn = 400

  def body(h_ref, wl_ref, bl_ref, wr_ref, br_ref, xl_ref, xr_ref):
    hb = h_ref[...]
    xl_ref[...] = (jnp.dot(hb, wl_ref[...], preferred_element_type=_F32,
                           precision=_PREC) + bl_ref[...])
    xr_ref[...] = (jnp.dot(hb, wr_ref[...], preferred_element_type=_F32,
                           precision=_PREC) + br_ref[...])

  return pl.pallas_call(
      body,
      grid=(n // bn,),
      in_specs=[
          pl.BlockSpec((bn, din), lambda i: (i, 0)),
          pl.BlockSpec((din, HC), lambda i: (0, 0)),
          pl.BlockSpec((1, HC), lambda i: (0, 0)),
          pl.BlockSpec((din, HC), lambda i: (0, 0)),
          pl.BlockSpec((1, HC), lambda i: (0, 0)),
      ],
      out_specs=[
          pl.BlockSpec((bn, HC), lambda i: (i, 0)),
          pl.BlockSpec((bn, HC), lambda i: (i, 0)),
      ],
      out_shape=[
          jax.ShapeDtypeStruct((n, HC), _F32),
          jax.ShapeDtypeStruct((n, HC), _F32),
      ],
  )(h, Wl, bl.reshape(1, HC), Wr, br.reshape(1, HC))


def _ef_tc(ea, We):
  be = 2000
  de = ea.shape[1]

  def body(ea_ref, we_ref, out_ref):
    out_ref[...] = jnp.dot(ea_ref[...], we_ref[...],
                           preferred_element_type=_F32, precision=_PREC)

  return pl.pallas_call(
      body,
      grid=(E // be,),
      in_specs=[
          pl.BlockSpec((be, de), lambda i: (i, 0)),
          pl.BlockSpec((de, HC), lambda i: (0, 0)),
      ],
      out_specs=pl.BlockSpec((be, HC), lambda i: (i, 0)),
      out_shape=jax.ShapeDtypeStruct((E, HC), _F32),
  )(ea, We)


def _combine_tc(p0, p1, bo):
  bn = 1000

  def body(p0_ref, p1_ref, bo_ref, out_ref):
    v = p0_ref[...] + p1_ref[...] + bo_ref[...]
    out_ref[...] = jnp.where(v >= 0.0, v, 0.01 * v)

  return pl.pallas_call(
      body,
      grid=(N // bn,),
      in_specs=[
          pl.BlockSpec((bn, C), lambda i: (i, 0)),
          pl.BlockSpec((bn, C), lambda i: (i, 0)),
          pl.BlockSpec((1, C), lambda i: (0, 0)),
      ],
      out_specs=pl.BlockSpec((bn, C), lambda i: (i, 0)),
      out_shape=jax.ShapeDtypeStruct((N, C), _F32),
  )(p0, p1, bo.reshape(1, C))


def _mlp_tc(h, D1W, D1b, D2W, D2b):
  bn = 400
  dd = D1W.shape[1]
  do = D2W.shape[1]

  def body(h_ref, w1_ref, b1_ref, w2_ref, b2_ref, out_ref):
    t = (jnp.dot(h_ref[...], w1_ref[...], preferred_element_type=_F32,
                 precision=_PREC) + b1_ref[...])
    out_ref[...] = (jnp.dot(t, w2_ref[...], preferred_element_type=_F32,
                            precision=_PREC) + b2_ref[...])

  return pl.pallas_call(
      body,
      grid=(N // bn,),
      in_specs=[
          pl.BlockSpec((bn, C), lambda i: (i, 0)),
          pl.BlockSpec((C, dd), lambda i: (0, 0)),
          pl.BlockSpec((1, dd), lambda i: (0, 0)),
          pl.BlockSpec((dd, do), lambda i: (0, 0)),
          pl.BlockSpec((1, do), lambda i: (0, 0)),
      ],
      out_specs=pl.BlockSpec((bn, do), lambda i: (i, 0)),
      out_shape=jax.ShapeDtypeStruct((N, do), _F32),
  )(h, D1W, D1b.reshape(1, dd), D2W, D2b.reshape(1, do))


def _gat_layer(h, src, dst, ef, Wl, bl, Wr, br, att, bo):
  xl, xr = _proj_tc(h, Wl, bl, Wr, br)
  logits = _sc_logits_kernel(xl, xr, ef, src, dst, att.reshape(HC))
  mpart = _sc_segmax_kernel(logits, dst)
  ex, spart = _sc_exsum_kernel(logits, dst, mpart)
  alpha = _sc_alpha_kernel(ex, dst, spart)
  outpart = _sc_message_kernel(xl, src, dst, alpha)
  return _combine_tc(outpart[0], outpart[1], bo)


def kernel(x, edge_index, edge_attr, Wl0, bl0, Wr0, br0, We0, att0, bo0,
           Wl1, bl1, Wr1, br1, We1, att1, bo1, Wl2, bl2, Wr2, br2, We2, att2,
           bo2, D1W, D1b, D2W, D2b):
  src = edge_index[0]
  dst = edge_index[1]
  h = x
  for (Wl, bl, Wr, br, We, att, bo) in (
      (Wl0, bl0, Wr0, br0, We0, att0, bo0),
      (Wl1, bl1, Wr1, br1, We1, att1, bo1),
      (Wl2, bl2, Wr2, br2, We2, att2, bo2)):
    ef = _ef_tc(edge_attr, We)
    h = _gat_layer(h, src, dst, ef, Wl, bl, Wr, br, att, bo)
  return _mlp_tc(h, D1W, D1b, D2W, D2b)
